# baseline (speedup 1.0000x reference)
import functools

import jax
import jax.numpy as jnp
from jax import lax
from jax.experimental import pallas as pl
from jax.experimental.pallas import tpu as pltpu
from jax.experimental.pallas import tpu_sc as plsc

_INFO = plsc.get_sparse_core_info()
_NC = _INFO.num_cores
_NS = _INFO.num_subcores
_NW = _NC * _NS
_BLK = 128
_NSLOT = 4


def _sc_gather_t(table, idx_t):
    J, I = idx_t.shape
    D = table.shape[1]
    FB = D // 8
    NB = I // _BLK
    n_units = J * NB
    upw = n_units // _NW
    assert upw % _NSLOT == 0 and upw >= 2 * _NSLOT
    n_tiles = J * FB * NB
    mesh = plsc.VectorSubcoreMesh(core_axis_name="c", subcore_axis_name="s")

    @functools.partial(
        pl.kernel,
        out_type=jax.ShapeDtypeStruct((n_tiles, 8, _BLK), jnp.float32),
        mesh=mesh,
        scratch_types=[
            pltpu.VMEM((_NSLOT, _BLK), jnp.int32),
            pltpu.VMEM((_NSLOT, _BLK, D), jnp.float32),
            pltpu.VMEM((2, FB, 8, _BLK), jnp.float32),
            [pltpu.SemaphoreType.DMA] * _NSLOT,
            [pltpu.SemaphoreType.DMA] * _NSLOT,
            [pltpu.SemaphoreType.DMA] * 2,
        ],
        compiler_params=pltpu.CompilerParams(
            use_tc_tiling_on_sc=False, needs_layout_passes=False),
    )
    def k(table_hbm, idx_hbm, out_hbm, idx_v, rows_v, tbuf, g, i, o):
        wid = lax.axis_index("s") * _NC + lax.axis_index("c")
        ubase = wid * upw
        iota16 = lax.iota(jnp.int32, 16)
        rowsel = [iota16 + 16 * blk for blk in range(_BLK // 16)]

        def unit_jb(u):
            ug = ubase + u
            return ug // NB, ug % NB

        def idx_slice(u):
            j, ib = unit_jb(u)
            return idx_hbm.at[j, pl.ds(ib * _BLK, _BLK)]

        def start_gather(s):
            pltpu.async_copy(
                table_hbm.at[idx_v.at[s]], rows_v.at[s], g[s])

        def wait_g(s):
            pltpu.make_async_copy(
                table_hbm.at[idx_v.at[s]], rows_v.at[s], g[s]).wait()

        def wait_i(s):
            pltpu.make_async_copy(idx_slice(0), idx_v.at[s], i[s]).wait()

        def wait_o(ts):
            for _ in range(FB):
                pltpu.make_async_copy(
                    tbuf.at[ts, 0], out_hbm.at[0], o[ts]).wait()

        def transpose(s, ts):
            nblk = _BLK // 16

            @pl.loop(0, FB)
            def _(fb):
                for fi in range(8):
                    f = fb * 8 + fi
                    col = jnp.full((16,), 0, jnp.int32) + f
                    vs = [plsc.load_gather(
                              rows_v.at[s], [rowsel[blk], col])
                          for blk in range(nblk)]
                    for blk in range(nblk):
                        tbuf[ts, fb, fi, pl.ds(16 * blk, 16)] = vs[blk]

        def emit_out(u, ts):
            j, ib = unit_jb(u)
            tb = j * (FB * NB) + ib
            for fb in range(FB):
                pltpu.async_copy(
                    tbuf.at[ts, fb], out_hbm.at[tb + fb * NB], o[ts])

        def step(u, s, ts):
            wait_g(s)

            def prefetch_idx():
                pltpu.async_copy(idx_slice(u + _NSLOT), idx_v.at[s], i[s])
                return None

            pl.when(u + _NSLOT < upw)(prefetch_idx)

            def next_gather():
                s3 = (s + _NSLOT - 1) % _NSLOT
                wait_i(s3)
                start_gather(s3)
                return None

            pl.when(u + _NSLOT - 1 < upw)(next_gather)

            def drain_out():
                wait_o(ts)
                return None

            pl.when(u >= 2)(drain_out)
            transpose(s, ts)
            emit_out(u, ts)

        pltpu.sync_copy(idx_slice(0), idx_v.at[0])
        pltpu.sync_copy(idx_slice(1), idx_v.at[1])
        pltpu.sync_copy(idx_slice(2), idx_v.at[2])
        pltpu.async_copy(idx_slice(3), idx_v.at[3], i[3])
        start_gather(0)
        start_gather(1)
        start_gather(2)

        @pl.loop(0, upw // _NSLOT)
        def _(h):
            for b in range(_NSLOT):
                u = _NSLOT * h + b
                step(u, b, b % 2)

        wait_o(0)
        wait_o(1)

    return k(table, idx_t)


def kernel(token_ids, embedding):
    I, J = token_ids.shape
    D = embedding.shape[1]
    idx_t = token_ids.T.astype(jnp.int32)
    out = _sc_gather_t(embedding, idx_t)
    FB, NB = D // 8, I // _BLK
    y = out.reshape(J, FB, NB, 8, _BLK)
    y = y.transpose(2, 4, 0, 1, 3)
    return y.reshape(I, J, D)


# speedup vs baseline: 1.0006x; 1.0006x over previous
import functools

import jax
import jax.numpy as jnp
from jax import lax
from jax.experimental import pallas as pl
from jax.experimental.pallas import tpu as pltpu
from jax.experimental.pallas import tpu_sc as plsc

_INFO = plsc.get_sparse_core_info()
_NC = _INFO.num_cores
_NS = _INFO.num_subcores
_NW = _NC * _NS
_BLK = 256
_NSLOT = 4


def _sc_gather_t(table, idx_flat, J, I):
    D = table.shape[1]
    FB = D // 8
    NBP = I // _BLK
    n_units = J * NBP
    upw = n_units // _NW
    assert upw % _NSLOT == 0
    bpw = upw * _BLK
    mesh = plsc.VectorSubcoreMesh(core_axis_name="c", subcore_axis_name="s")

    @functools.partial(
        pl.kernel,
        out_type=jax.ShapeDtypeStruct((J, FB, NBP, _BLK * 8), jnp.float32),
        mesh=mesh,
        scratch_types=[
            pltpu.VMEM((bpw,), jnp.int32),
            pltpu.VMEM((_NSLOT, _BLK, D), jnp.float32),
            pltpu.VMEM((2, FB, _BLK * 8), jnp.float32),
            pltpu.SemaphoreType.DMA,
            [pltpu.SemaphoreType.DMA] * _NSLOT,
            [pltpu.SemaphoreType.DMA] * 2,
        ],
        compiler_params=pltpu.CompilerParams(
            use_tc_tiling_on_sc=False, needs_layout_passes=False),
    )
    def k(table_hbm, idx_hbm, out_hbm, idx_v, rows_v, tbuf, isem, g, o):
        wid = lax.axis_index("s") * _NC + lax.axis_index("c")
        ubase = wid * upw
        iota16 = lax.iota(jnp.int32, 16)
        rowsel = [iota16 + 16 * grp for grp in range(_BLK // 16)]

        def idx_ref(u):
            return idx_v.at[pl.ds(u * _BLK, _BLK)]

        def start_gather(u, s):
            pltpu.async_copy(
                table_hbm.at[idx_ref(u)], rows_v.at[s], g[s])

        def wait_g(u, s):
            pltpu.make_async_copy(
                table_hbm.at[idx_ref(u)], rows_v.at[s], g[s]).wait()

        def out_slice(u):
            ug = ubase + u
            j = ug // NBP
            ibp = ug % NBP
            return out_hbm.at[j, :, ibp]

        def wait_o(ts):
            pltpu.make_async_copy(
                tbuf.at[ts], out_slice(0), o[ts]).wait()

        def transpose(s, ts):
            @pl.loop(0, FB)
            def _(fb):
                for half in range(2):
                    for fi in range(8):
                        f = fb * 8 + fi
                        col = jnp.full((16,), 0, jnp.int32) + f
                        base = half * 1024 + fi * 128
                        vs = [plsc.load_gather(
                                  rows_v.at[s],
                                  [rowsel[half * 8 + blk], col])
                              for blk in range(8)]
                        for blk in range(8):
                            tbuf[ts, fb, pl.ds(base + 16 * blk, 16)] = vs[blk]

        def step(u, s, ts):
            wait_g(u, s)

            def next_gather():
                s3 = (s + _NSLOT - 1) % _NSLOT
                start_gather(u + _NSLOT - 1, s3)
                return None

            pl.when(u + _NSLOT - 1 < upw)(next_gather)

            def drain_out():
                wait_o(ts)
                return None

            pl.when(u >= 2)(drain_out)
            transpose(s, ts)
            pltpu.async_copy(tbuf.at[ts], out_slice(u), o[ts])

        pltpu.sync_copy(idx_hbm.at[pl.ds(ubase * _BLK, bpw)], idx_v)
        for s0 in range(_NSLOT - 1):
            start_gather(s0, s0)

        @pl.loop(0, upw // _NSLOT)
        def _(h):
            for b in range(_NSLOT):
                u = _NSLOT * h + b
                step(u, b, b % 2)

        wait_o(0)
        wait_o(1)

    return k(table, idx_flat)


def kernel(token_ids, embedding):
    I, J = token_ids.shape
    D = embedding.shape[1]
    idx_flat = token_ids.T.astype(jnp.int32).reshape(-1)
    out = _sc_gather_t(embedding, idx_flat, J, I)
    FB, NBP = D // 8, I // _BLK
    y = out.reshape(J, FB, NBP, 2, 8, 128)
    y = y.transpose(2, 3, 5, 0, 1, 4)
    return y.reshape(I, J, D)


# speedup vs baseline: 2.0014x; 2.0002x over previous
import functools

import jax
import jax.numpy as jnp
from jax import lax
from jax.experimental import pallas as pl
from jax.experimental.pallas import tpu as pltpu
from jax.experimental.pallas import tpu_sc as plsc

_INFO = plsc.get_sparse_core_info()
_NC = _INFO.num_cores
_NS = _INFO.num_subcores
_NW = _NC * _NS
_BLK = 256
_NSLOT = 4


def _sc_gather_t(table, idx_flat, J, I):
    D = table.shape[1]
    FB = D // 8
    NBP = I // _BLK
    n_units = J * NBP
    upw = n_units // _NW
    assert upw % _NSLOT == 0
    bpw = upw * _BLK
    mesh = plsc.VectorSubcoreMesh(core_axis_name="c", subcore_axis_name="s")

    @functools.partial(
        pl.kernel,
        out_type=jax.ShapeDtypeStruct((J, FB, NBP, _BLK * 8), jnp.float32),
        mesh=mesh,
        scratch_types=[
            pltpu.VMEM((bpw,), jnp.int32),
            pltpu.VMEM((_NSLOT, _BLK, D), jnp.float32),
            pltpu.VMEM((2, FB * _BLK * 8), jnp.float32),
            pltpu.SemaphoreType.DMA,
            [pltpu.SemaphoreType.DMA] * _NSLOT,
            [pltpu.SemaphoreType.DMA] * 2,
        ],
        compiler_params=pltpu.CompilerParams(
            use_tc_tiling_on_sc=False, needs_layout_passes=False),
    )
    def k(table_hbm, idx_hbm, out_hbm, idx_v, rows_v, tbuf, isem, g, o):
        wid = lax.axis_index("s") * _NC + lax.axis_index("c")
        ubase = wid * upw
        iota16 = lax.iota(jnp.int32, 16)
        diag = [(iota16 + d) & 15 for d in range(16)]
        scol = [(dv >> 3) * 2048 + (dv & 7) * 128 + iota16 for dv in diag]

        def idx_ref(u):
            return idx_v.at[pl.ds(u * _BLK, _BLK)]

        def start_gather(u, s):
            pltpu.async_copy(
                table_hbm.at[idx_ref(u)], rows_v.at[s], g[s])

        def wait_g(u, s):
            pltpu.make_async_copy(
                table_hbm.at[idx_ref(u)], rows_v.at[s], g[s]).wait()

        def out_slice(u, fb):
            ug = ubase + u
            j = ug // NBP
            ibp = ug % NBP
            return out_hbm.at[j, fb, ibp]

        def wait_o(ts):
            for _ in range(FB):
                pltpu.make_async_copy(
                    tbuf.at[ts, pl.ds(0, _BLK * 8)], out_slice(0, 0),
                    o[ts]).wait()

        def transpose(s, ts):
            @pl.loop(0, _BLK // 16)
            def _(t0g):
                t0 = t0g * 16
                rvec = iota16 + t0
                tpart = (t0 >> 7) * 1024 + (t0 & 127)
                for f0 in (0, 16, 32, 48):
                    sbase = (f0 >> 3) * 2048 + tpart
                    for dg in range(2):
                        vs = [plsc.load_gather(
                                  rows_v.at[s], [rvec, diag[d] + f0])
                              for d in range(8 * dg, 8 * dg + 8)]
                        for q, d in enumerate(range(8 * dg, 8 * dg + 8)):
                            plsc.store_scatter(
                                tbuf.at[ts], [scol[d] + sbase], vs[q])

        def step(u, s, ts):
            wait_g(u, s)

            def next_gather():
                s3 = (s + _NSLOT - 1) % _NSLOT
                start_gather(u + _NSLOT - 1, s3)
                return None

            pl.when(u + _NSLOT - 1 < upw)(next_gather)

            def drain_out():
                wait_o(ts)
                return None

            pl.when(u >= 2)(drain_out)
            transpose(s, ts)
            for fb in range(FB):
                pltpu.async_copy(
                    tbuf.at[ts, pl.ds(fb * _BLK * 8, _BLK * 8)],
                    out_slice(u, fb), o[ts])

        pltpu.sync_copy(idx_hbm.at[pl.ds(ubase * _BLK, bpw)], idx_v)
        for s0 in range(_NSLOT - 1):
            start_gather(s0, s0)

        @pl.loop(0, upw // _NSLOT)
        def _(h):
            for b in range(_NSLOT):
                u = _NSLOT * h + b
                step(u, b, b % 2)

        wait_o(0)
        wait_o(1)

    return k(table, idx_flat)


def kernel(token_ids, embedding):
    I, J = token_ids.shape
    D = embedding.shape[1]
    idx_flat = token_ids.T.astype(jnp.int32).reshape(-1)
    out = _sc_gather_t(embedding, idx_flat, J, I)
    FB, NBP = D // 8, I // _BLK
    y = out.reshape(J, FB, NBP, 2, 8, 128)
    y = y.transpose(2, 3, 5, 0, 1, 4)
    return y.reshape(I, J, D)


# speedup vs baseline: 2.0020x; 1.0003x over previous
import functools

import jax
import jax.numpy as jnp
from jax import lax
from jax.experimental import pallas as pl
from jax.experimental.pallas import tpu as pltpu
from jax.experimental.pallas import tpu_sc as plsc

_INFO = plsc.get_sparse_core_info()
_NC = _INFO.num_cores
_NS = _INFO.num_subcores
_NW = _NC * _NS
_BLK = 256
_NSLOT = 4


def _sc_gather_t(table, idx_flat, J, I):
    D = table.shape[1]
    FB = D // 8
    NBP = I // _BLK
    n_units = J * NBP
    upw = n_units // _NW
    assert upw % _NSLOT == 0
    bpw = upw * _BLK
    mesh = plsc.VectorSubcoreMesh(core_axis_name="c", subcore_axis_name="s")

    @functools.partial(
        pl.kernel,
        out_type=jax.ShapeDtypeStruct((J, FB, NBP, _BLK * 8), jnp.float32),
        mesh=mesh,
        scratch_types=[
            pltpu.VMEM((bpw,), jnp.int32),
            pltpu.VMEM((_NSLOT, _BLK, D), jnp.float32),
            pltpu.VMEM((2, FB * _BLK * 8), jnp.float32),
            pltpu.SemaphoreType.DMA,
            [pltpu.SemaphoreType.DMA] * _NSLOT,
            [pltpu.SemaphoreType.DMA] * 2,
        ],
        compiler_params=pltpu.CompilerParams(
            use_tc_tiling_on_sc=False, needs_layout_passes=False),
    )
    def k(table_hbm, idx_hbm, out_hbm, idx_v, rows_v, tbuf, isem, g, o):
        wid = lax.axis_index("s") * _NC + lax.axis_index("c")
        ubase = wid * upw
        iota16 = lax.iota(jnp.int32, 16)
        diag = [(iota16 + d) & 15 for d in range(16)]
        scol = [(dv >> 3) * 2048 + (dv & 7) * 128 + iota16 for dv in diag]

        def idx_ref(u):
            return idx_v.at[pl.ds(u * _BLK, _BLK)]

        def start_gather(u, s):
            pltpu.async_copy(
                table_hbm.at[idx_ref(u)], rows_v.at[s], g[s])

        def wait_g(u, s):
            pltpu.make_async_copy(
                table_hbm.at[idx_ref(u)], rows_v.at[s], g[s]).wait()

        def out_slice(u, fb):
            ug = ubase + u
            j = ug // NBP
            ibp = ug % NBP
            return out_hbm.at[j, fb, ibp]

        def wait_o(ts):
            for _ in range(FB):
                pltpu.make_async_copy(
                    tbuf.at[ts, pl.ds(0, _BLK * 8)], out_slice(0, 0),
                    o[ts]).wait()

        def transpose(s, ts):
            @pl.loop(0, _BLK // 16)
            def _(t0g):
                t0 = t0g * 16
                rvec = iota16 + t0
                tpart = (t0 >> 7) * 1024 + (t0 & 127)
                for f0 in (0, 16, 32, 48):
                    sbase = (f0 >> 3) * 2048 + tpart
                    for dg in range(2):
                        vs = [plsc.load_gather(
                                  rows_v.at[s], [rvec, diag[d] + f0])
                              for d in range(8 * dg, 8 * dg + 8)]
                        for q, d in enumerate(range(8 * dg, 8 * dg + 8)):
                            plsc.store_scatter(
                                tbuf.at[ts], [scol[d] + sbase], vs[q])

        def step(u, s, ts):
            wait_g(u, s)

            def next_gather():
                s3 = (s + _NSLOT - 1) % _NSLOT
                start_gather(u + _NSLOT - 1, s3)
                return None

            pl.when(u + _NSLOT - 1 < upw)(next_gather)

            def drain_out():
                wait_o(ts)
                return None

            pl.when(u >= 2)(drain_out)
            transpose(s, ts)
            for fb in range(FB):
                pltpu.async_copy(
                    tbuf.at[ts, pl.ds(fb * _BLK * 8, _BLK * 8)],
                    out_slice(u, fb), o[ts])

        pltpu.sync_copy(idx_hbm.at[pl.ds(ubase * _BLK, bpw)], idx_v)
        for s0 in range(_NSLOT - 1):
            start_gather(s0, s0)

        @pl.loop(0, upw // _NSLOT)
        def _(h):
            for b in range(_NSLOT):
                u = _NSLOT * h + b
                step(u, b, b % 2)

        wait_o(0)
        wait_o(1)

    return k(table, idx_flat)


def kernel(token_ids, embedding):
    I, J = token_ids.shape
    D = embedding.shape[1]
    V = embedding.shape[0]
    idx_flat = token_ids.T.astype(jnp.int32).reshape(-1)
    emb2 = lax.optimization_barrier(embedding.reshape(V // 2, 2 * D))
    emb_lin = emb2.reshape(V, D)
    out = _sc_gather_t(emb_lin, idx_flat, J, I)
    FB, NBP = D // 8, I // _BLK
    y = out.reshape(J, FB, NBP, 2, 8, 128)
    y = y.transpose(2, 3, 5, 0, 1, 4)
    return y.reshape(I, J, D)
